# baseline (speedup 1.0000x reference)
import functools

import jax
import jax.numpy as jnp
from jax import lax
from jax.experimental import pallas as pl
from jax.experimental.pallas import tpu as pltpu
from jax.experimental.pallas import tpu_sc as plsc

_NC = 2
_NS = 16
_LANES = 16
_NW = _NC * _NS

_B, _L, _W, _H = 64, 80, 80, 40
_N = 16384
_NPTS = _B * _N
_PPW = _NPTS // _NW
_CHUNK = 2048
_ITERS = _PPW // _CHUNK
_GROUPS = _CHUNK // _LANES
_NCORN = 8

_GRID_RES = 0.1
_INV_RES = 1.0 / _GRID_RES


def _floor_to_int(q):
  t = q.astype(jnp.int32)
  tf = t.astype(jnp.float32)
  adj = (tf > q)
  ti = jnp.where(adj, t - 1, t)
  return ti, jnp.where(adj, tf - 1.0, tf)


def _tec_body(vox_hbm, xs_hbm, ys_hbm, zs_hbm, hs_hbm, out_hbm,
              x_v, y_v, z_v, h_v, idx_v, w_v, val_v, part_v, sem):
  wid = lax.axis_index("s") * _NC + lax.axis_index("c")
  base_pt = wid * _PPW

  def chunk_body(it, acc):
    off = base_pt + it * _CHUNK
    pltpu.sync_copy(xs_hbm.at[pl.ds(off, _CHUNK)], x_v)
    pltpu.sync_copy(ys_hbm.at[pl.ds(off, _CHUNK)], y_v)
    pltpu.sync_copy(zs_hbm.at[pl.ds(off, _CHUNK)], z_v)
    pltpu.sync_copy(hs_hbm.at[pl.ds(off, _CHUNK)], h_v)

    bbase = (off // _N) * (_L * _W * _H)

    def group_body(i, _):
      s = i * _LANES
      px = x_v[pl.ds(s, _LANES)]
      py = y_v[pl.ds(s, _LANES)]
      pz = z_v[pl.ds(s, _LANES)]
      hh = h_v[pl.ds(s, _LANES)]

      x = px + 4.0
      y = py + 4.0
      z = pz + hh * 0.5

      xq = x * _INV_RES
      yq = y * _INV_RES
      zq = z * _INV_RES
      xi, xf = _floor_to_int(xq)
      yi, yf = _floor_to_int(yq)
      zi, zf = _floor_to_int(zq)
      tx = (x - xf * _GRID_RES) * _INV_RES
      ty = (y - yf * _GRID_RES) * _INV_RES
      tz = (z - zf * _GRID_RES) * _INV_RES

      zero = jnp.zeros((_LANES,), jnp.int32)
      xmin = jnp.clip(xi, zero, _L - 1)
      xmax = jnp.clip(xi + 1, zero, _L - 1)
      ymin = jnp.clip(yi, zero, _W - 1)
      ymax = jnp.clip(yi + 1, zero, _W - 1)
      zmin = jnp.clip(zi, zero, _H - 1)
      zmax = jnp.clip(zi + 1, zero, _H - 1)

      axmin = bbase + xmin * (_W * _H)
      axmax = bbase + xmax * (_W * _H)
      bymin = ymin * _H
      bymax = ymax * _H

      ux = 1.0 - tx
      uy = 1.0 - ty
      uz = 1.0 - tz
      wxy_pp = tx * ty
      wxy_pm = tx * uy
      wxy_mp = ux * ty
      wxy_mm = ux * uy

      base_i = i * (_NCORN * _LANES)
      idx_v[pl.ds(base_i + 0 * _LANES, _LANES)] = axmax + bymax + zmax
      idx_v[pl.ds(base_i + 1 * _LANES, _LANES)] = axmax + bymax + zmin
      idx_v[pl.ds(base_i + 2 * _LANES, _LANES)] = axmax + bymin + zmax
      idx_v[pl.ds(base_i + 3 * _LANES, _LANES)] = axmax + bymin + zmin
      idx_v[pl.ds(base_i + 4 * _LANES, _LANES)] = axmin + bymax + zmax
      idx_v[pl.ds(base_i + 5 * _LANES, _LANES)] = axmin + bymax + zmin
      idx_v[pl.ds(base_i + 6 * _LANES, _LANES)] = axmin + bymin + zmax
      idx_v[pl.ds(base_i + 7 * _LANES, _LANES)] = axmin + bymin + zmin

      w_v[pl.ds(base_i + 0 * _LANES, _LANES)] = wxy_pp * tz
      w_v[pl.ds(base_i + 1 * _LANES, _LANES)] = wxy_pp * uz
      w_v[pl.ds(base_i + 2 * _LANES, _LANES)] = wxy_pm * tz
      w_v[pl.ds(base_i + 3 * _LANES, _LANES)] = wxy_pm * uz
      w_v[pl.ds(base_i + 4 * _LANES, _LANES)] = wxy_mp * tz
      w_v[pl.ds(base_i + 5 * _LANES, _LANES)] = wxy_mp * uz
      w_v[pl.ds(base_i + 6 * _LANES, _LANES)] = wxy_mm * tz
      w_v[pl.ds(base_i + 7 * _LANES, _LANES)] = wxy_mm * uz
      return _

    lax.fori_loop(0, _GROUPS, group_body, 0)

    pltpu.async_copy(vox_hbm.at[idx_v], val_v, sem).wait()

    def comb_body(i, acc_in):
      base_i = i * (_NCORN * _LANES)
      sdf = (val_v[pl.ds(base_i + 0 * _LANES, _LANES)]
             * w_v[pl.ds(base_i + 0 * _LANES, _LANES)])
      sdf = sdf + (val_v[pl.ds(base_i + 1 * _LANES, _LANES)]
                   * w_v[pl.ds(base_i + 1 * _LANES, _LANES)])
      sdf = sdf + (val_v[pl.ds(base_i + 2 * _LANES, _LANES)]
                   * w_v[pl.ds(base_i + 2 * _LANES, _LANES)])
      sdf = sdf + (val_v[pl.ds(base_i + 3 * _LANES, _LANES)]
                   * w_v[pl.ds(base_i + 3 * _LANES, _LANES)])
      sdf = sdf + (val_v[pl.ds(base_i + 4 * _LANES, _LANES)]
                   * w_v[pl.ds(base_i + 4 * _LANES, _LANES)])
      sdf = sdf + (val_v[pl.ds(base_i + 5 * _LANES, _LANES)]
                   * w_v[pl.ds(base_i + 5 * _LANES, _LANES)])
      sdf = sdf + (val_v[pl.ds(base_i + 6 * _LANES, _LANES)]
                   * w_v[pl.ds(base_i + 6 * _LANES, _LANES)])
      sdf = sdf + (val_v[pl.ds(base_i + 7 * _LANES, _LANES)]
                   * w_v[pl.ds(base_i + 7 * _LANES, _LANES)])
      ax = jnp.abs(sdf)
      hub = jnp.where(ax < 1.0, 0.5 * sdf * sdf, ax - 0.5)
      return acc_in + hub

    return lax.fori_loop(0, _GROUPS, comb_body, acc)

  acc = lax.fori_loop(0, _ITERS, chunk_body, jnp.zeros((_LANES,), jnp.float32))
  part_v[...] = acc
  pltpu.sync_copy(part_v, out_hbm.at[wid])


@jax.jit
def kernel(voxels, pts_centroid, height_gt):
  vox_flat = voxels.reshape(-1)
  xs = pts_centroid[..., 0].reshape(-1)
  ys = pts_centroid[..., 1].reshape(-1)
  zs = pts_centroid[..., 2].reshape(-1)
  hs = height_gt.reshape(-1)

  mesh = plsc.VectorSubcoreMesh(
      core_axis_name="c", subcore_axis_name="s",
      num_cores=_NC, num_subcores=_NS)
  kfn = pl.kernel(
      _tec_body,
      out_type=jax.ShapeDtypeStruct((_NW, _LANES), jnp.float32),
      mesh=mesh,
      scratch_types=[
          pltpu.VMEM((_CHUNK,), jnp.float32),
          pltpu.VMEM((_CHUNK,), jnp.float32),
          pltpu.VMEM((_CHUNK,), jnp.float32),
          pltpu.VMEM((_CHUNK,), jnp.float32),
          pltpu.VMEM((_NCORN * _CHUNK,), jnp.int32),
          pltpu.VMEM((_NCORN * _CHUNK,), jnp.float32),
          pltpu.VMEM((_NCORN * _CHUNK,), jnp.float32),
          pltpu.VMEM((_LANES,), jnp.float32),
          pltpu.SemaphoreType.DMA,
      ],
  )
  partials = kfn(vox_flat, xs, ys, zs, hs)
  return jnp.sum(partials) / _NPTS
